# baseline (speedup 1.0000x reference)
import functools

import jax
import jax.numpy as jnp
from jax import lax
from jax.experimental import pallas as pl
from jax.experimental.pallas import tpu as pltpu
from jax.experimental.pallas import tpu_sc as plsc

NUM_EMOTIONS = 1000
EMB_DIM = 128
BATCH = 16384


SC_ROWS = 8192
NUM_CORES = 2
NUM_SUBCORES = 16
NUM_WORKERS = NUM_CORES * NUM_SUBCORES
B_PER_W = SC_ROWS // NUM_WORKERS


def _make_sc_gather():
    mesh = plsc.VectorSubcoreMesh(core_axis_name="c", subcore_axis_name="s")

    @functools.partial(
        pl.kernel,
        mesh=mesh,
        out_type=jax.ShapeDtypeStruct((SC_ROWS, EMB_DIM), jnp.float32),
        scratch_types=[
            pltpu.VMEM((B_PER_W,), jnp.int32),
            pltpu.VMEM((B_PER_W, EMB_DIM), jnp.float32),
            pltpu.SemaphoreType.DMA,
        ],
    )
    def sc_gather(table_hbm, idx_hbm, out_hbm, idx_v, rows_v, sem):
        wid = lax.axis_index("s") * NUM_CORES + lax.axis_index("c")
        base = wid * B_PER_W
        pltpu.sync_copy(idx_hbm.at[pl.ds(base, B_PER_W)], idx_v)
        pltpu.async_copy(table_hbm.at[idx_v], rows_v, sem).wait()
        pltpu.sync_copy(rows_v, out_hbm.at[pl.ds(base, B_PER_W)])

    return sc_gather


_sc_gather = _make_sc_gather()


BLK = 2048
SC_BLKS = SC_ROWS // BLK
TC_BLKS = (BATCH - SC_ROWS) // BLK


def _tc_body(idx_ref, t_ref, o_ref):
    idx = idx_ref[...]
    b = jnp.broadcast_to(idx[None, :], (NUM_EMOTIONS, BLK))
    iota = jax.lax.broadcasted_iota(jnp.int32, (NUM_EMOTIONS, BLK), 0)
    oh_t = (b == iota).astype(jnp.bfloat16)
    w = t_ref[...].astype(jnp.bfloat16)
    o_ref[...] = jax.lax.dot_general(
        oh_t, w, (((0,), (0,)), ((), ())),
        preferred_element_type=jnp.float32)


def _tc_gather(idx, table):
    return pl.pallas_call(
        _tc_body,
        out_shape=jax.ShapeDtypeStruct((BATCH, EMB_DIM), jnp.float32),
        grid=(TC_BLKS,),
        in_specs=[
            pl.BlockSpec((BLK,), lambda i: (i + SC_BLKS,)),
            pl.BlockSpec((NUM_EMOTIONS, EMB_DIM), lambda i: (0, 0)),
        ],
        out_specs=pl.BlockSpec((BLK, EMB_DIM), lambda i: (i + SC_BLKS, 0)),
    )(idx, table)


def kernel(emotion_id, table):
    idx = emotion_id.astype(jnp.int32)
    out_sc = _sc_gather(table, idx)
    out_tc = _tc_gather(idx, table)
    return lax.dynamic_update_slice(out_tc, out_sc, (0, 0))


# speedup vs baseline: 1.0626x; 1.0626x over previous
import functools

import jax
import jax.numpy as jnp
from jax import lax
from jax.experimental import pallas as pl
from jax.experimental.pallas import tpu as pltpu
from jax.experimental.pallas import tpu_sc as plsc

NUM_EMOTIONS = 1000
EMB_DIM = 128
BATCH = 16384


SC_ROWS = 6144
NUM_CORES = 2
NUM_SUBCORES = 16
NUM_WORKERS = NUM_CORES * NUM_SUBCORES
B_PER_W = SC_ROWS // NUM_WORKERS


def _make_sc_gather():
    mesh = plsc.VectorSubcoreMesh(core_axis_name="c", subcore_axis_name="s")

    @functools.partial(
        pl.kernel,
        mesh=mesh,
        out_type=jax.ShapeDtypeStruct((SC_ROWS, EMB_DIM), jnp.float32),
        scratch_types=[
            pltpu.VMEM((B_PER_W,), jnp.int32),
            pltpu.VMEM((B_PER_W, EMB_DIM), jnp.float32),
            pltpu.SemaphoreType.DMA,
        ],
    )
    def sc_gather(table_hbm, idx_hbm, out_hbm, idx_v, rows_v, sem):
        wid = lax.axis_index("s") * NUM_CORES + lax.axis_index("c")
        base = wid * B_PER_W
        pltpu.sync_copy(idx_hbm.at[pl.ds(base, B_PER_W)], idx_v)
        pltpu.async_copy(table_hbm.at[idx_v], rows_v, sem).wait()
        pltpu.sync_copy(rows_v, out_hbm.at[pl.ds(base, B_PER_W)])

    return sc_gather


_sc_gather = _make_sc_gather()


BLK = 2048
SC_BLKS = SC_ROWS // BLK
TC_BLKS = (BATCH - SC_ROWS) // BLK


def _tc_body(idx_ref, t_ref, o_ref):
    idx = idx_ref[...]
    b = jnp.broadcast_to(idx[None, :], (NUM_EMOTIONS, BLK))
    iota = jax.lax.broadcasted_iota(jnp.int32, (NUM_EMOTIONS, BLK), 0)
    oh_t = (b == iota).astype(jnp.bfloat16)
    w = t_ref[...].astype(jnp.bfloat16)
    o_ref[...] = jax.lax.dot_general(
        oh_t, w, (((0,), (0,)), ((), ())),
        preferred_element_type=jnp.float32)


def _tc_gather(idx, table):
    return pl.pallas_call(
        _tc_body,
        out_shape=jax.ShapeDtypeStruct((BATCH, EMB_DIM), jnp.float32),
        grid=(TC_BLKS,),
        in_specs=[
            pl.BlockSpec((BLK,), lambda i: (i + SC_BLKS,)),
            pl.BlockSpec((NUM_EMOTIONS, EMB_DIM), lambda i: (0, 0)),
        ],
        out_specs=pl.BlockSpec((BLK, EMB_DIM), lambda i: (i + SC_BLKS, 0)),
    )(idx, table)


def kernel(emotion_id, table):
    idx = emotion_id.astype(jnp.int32)
    out_sc = _sc_gather(table, idx)
    out_tc = _tc_gather(idx, table)
    return lax.dynamic_update_slice(out_tc, out_sc, (0, 0))
